# Initial kernel scaffold; baseline (speedup 1.0000x reference)
import jax
import jax.numpy as jnp
from jax.experimental import pallas as pl


def kernel(patches):
    raise NotImplementedError("write your pallas kernel here")


# speedup vs baseline: 2.0210x; 2.0210x over previous
import functools

import jax
import jax.numpy as jnp
import numpy as np
from jax import lax
from jax.experimental import pallas as pl
from jax.experimental.pallas import tpu as pltpu
from jax.experimental.pallas import tpu_sc as plsc

RATIO = 0.5
LANES = 16


def _rotl(x, r):
    return ((x << np.uint32(r)) | (x >> np.uint32(32 - r))).astype(np.uint32)


def _threefry2x32(k0, k1, x0, x1):
    rot = [[13, 15, 26, 6], [17, 29, 16, 24]]
    ks = [np.uint32(k0), np.uint32(k1),
          np.uint32(np.uint32(k0) ^ np.uint32(k1) ^ np.uint32(0x1BD11BDA))]
    x0 = (x0 + ks[0]).astype(np.uint32)
    x1 = (x1 + ks[1]).astype(np.uint32)
    for i in range(5):
        for r in rot[i % 2]:
            x0 = (x0 + x1).astype(np.uint32)
            x1 = _rotl(x1, r)
            x1 = (x1 ^ x0).astype(np.uint32)
        x0 = (x0 + ks[(i + 1) % 3]).astype(np.uint32)
        x1 = (x1 + ks[(i + 2) % 3] + np.uint32(i + 1)).astype(np.uint32)
    return x0, x1


@functools.lru_cache(maxsize=None)
def _plan(B, L, len_keep):
    size = B * L
    c64 = np.arange(size, dtype=np.uint64)
    hi = (c64 >> np.uint64(32)).astype(np.uint32)
    lo = (c64 & np.uint64(0xFFFFFFFF)).astype(np.uint32)
    b0, b1 = _threefry2x32(0, 1, hi, lo)
    bits = (b0 ^ b1).reshape(B, L)
    noise = ((bits >> np.uint32(9)) | np.uint32(0x3F800000)).view(np.float32) - 1.0
    shuffle = np.argsort(noise, axis=1, kind="stable")
    restore = np.argsort(shuffle, axis=1, kind="stable")
    keep = shuffle[:, :len_keep]
    return keep.astype(np.int32), restore.astype(np.int32)


def _build_sc_call(B, L, D, len_keep, n_chunks, chunk):
    info = plsc.get_sparse_core_info()
    nc, ns = info.num_cores, info.num_subcores
    nw = nc * ns
    assert B == nw, (B, nw)
    mesh = plsc.VectorSubcoreMesh(core_axis_name="c", subcore_axis_name="s")

    @functools.partial(
        pl.kernel,
        mesh=mesh,
        out_type=(
            jax.ShapeDtypeStruct((B * len_keep, D), jnp.float32),
            jax.ShapeDtypeStruct((B, L), jnp.float32),
            jax.ShapeDtypeStruct((B, L), jnp.int32),
        ),
        scratch_types=[
            pltpu.VMEM((n_chunks, chunk), jnp.int32),
            pltpu.VMEM((chunk, D), jnp.float32),
            pltpu.VMEM((chunk, D), jnp.float32),
            pltpu.VMEM((L,), jnp.int32),
            pltpu.VMEM((L,), jnp.float32),
            pltpu.SemaphoreType.DMA,
            pltpu.SemaphoreType.DMA,
        ],
    )
    def sc_call(table_hbm, gidx_hbm, restore_hbm,
                out_hbm, mask_hbm, restore_out_hbm,
                idx_v, buf0, buf1, rest_v, mask_v, sem0, sem1):
        wid = lax.axis_index("s") * nc + lax.axis_index("c")
        bufs = (buf0, buf1)
        sems = (sem0, sem1)

        pltpu.sync_copy(gidx_hbm.at[wid], idx_v)
        copies = [None] * n_chunks
        copies[0] = pltpu.async_copy(table_hbm.at[idx_v.at[0]], bufs[0], sems[0])

        pltpu.sync_copy(restore_hbm.at[wid], rest_v)
        for i in range(L // LANES):
            r = rest_v[pl.ds(i * LANES, LANES)]
            mask_v[pl.ds(i * LANES, LANES)] = jnp.where(
                r >= len_keep, jnp.float32(1.0), jnp.float32(0.0))
        pltpu.sync_copy(rest_v, restore_out_hbm.at[wid])
        pltpu.sync_copy(mask_v, mask_hbm.at[wid])

        base = wid * len_keep
        for c in range(n_chunks):
            if c + 1 < n_chunks:
                copies[c + 1] = pltpu.async_copy(
                    table_hbm.at[idx_v.at[c + 1]], bufs[(c + 1) % 2],
                    sems[(c + 1) % 2])
            copies[c].wait()
            pltpu.sync_copy(bufs[c % 2],
                            out_hbm.at[pl.ds(base + c * chunk, chunk)])

    return sc_call


def kernel(patches):
    B, L, D = patches.shape
    len_keep = int(L * (1 - RATIO))
    keep_np, restore_np = _plan(B, L, len_keep)

    n_chunks, chunk = 8, len_keep // 8
    gidx_np = (np.arange(B, dtype=np.int64)[:, None] * L
               + keep_np).astype(np.int32).reshape(B, n_chunks, chunk)

    sc_call = _build_sc_call(B, L, D, len_keep, n_chunks, chunk)
    table = patches.reshape(B * L, D)
    out, mask, restore = sc_call(table, jnp.asarray(gidx_np),
                                 jnp.asarray(restore_np))
    return out.reshape(B, len_keep, D), mask, restore


# speedup vs baseline: 2.0260x; 1.0025x over previous
import functools

import jax
import jax.numpy as jnp
import numpy as np
from jax import lax
from jax.experimental import pallas as pl
from jax.experimental.pallas import tpu as pltpu
from jax.experimental.pallas import tpu_sc as plsc

RATIO = 0.5
LANES = 16


def _rotl(x, r):
    return ((x << np.uint32(r)) | (x >> np.uint32(32 - r))).astype(np.uint32)


def _threefry2x32(k0, k1, x0, x1):
    rot = [[13, 15, 26, 6], [17, 29, 16, 24]]
    ks = [np.uint32(k0), np.uint32(k1),
          np.uint32(np.uint32(k0) ^ np.uint32(k1) ^ np.uint32(0x1BD11BDA))]
    x0 = (x0 + ks[0]).astype(np.uint32)
    x1 = (x1 + ks[1]).astype(np.uint32)
    for i in range(5):
        for r in rot[i % 2]:
            x0 = (x0 + x1).astype(np.uint32)
            x1 = _rotl(x1, r)
            x1 = (x1 ^ x0).astype(np.uint32)
        x0 = (x0 + ks[(i + 1) % 3]).astype(np.uint32)
        x1 = (x1 + ks[(i + 2) % 3] + np.uint32(i + 1)).astype(np.uint32)
    return x0, x1


@functools.lru_cache(maxsize=None)
def _plan(B, L, len_keep):
    size = B * L
    c64 = np.arange(size, dtype=np.uint64)
    hi = (c64 >> np.uint64(32)).astype(np.uint32)
    lo = (c64 & np.uint64(0xFFFFFFFF)).astype(np.uint32)
    b0, b1 = _threefry2x32(0, 1, hi, lo)
    bits = (b0 ^ b1).reshape(B, L)
    noise = ((bits >> np.uint32(9)) | np.uint32(0x3F800000)).view(np.float32) - 1.0
    shuffle = np.argsort(noise, axis=1, kind="stable")
    restore = np.argsort(shuffle, axis=1, kind="stable")
    keep = shuffle[:, :len_keep]
    return keep.astype(np.int32), restore.astype(np.int32)


def _build_sc_call(B, L, D, len_keep, n_chunks, chunk):
    info = plsc.get_sparse_core_info()
    nc, ns = info.num_cores, info.num_subcores
    nw = nc * ns
    assert B == nw, (B, nw)
    mesh = plsc.VectorSubcoreMesh(core_axis_name="c", subcore_axis_name="s")

    @functools.partial(
        pl.kernel,
        mesh=mesh,
        out_type=(
            jax.ShapeDtypeStruct((B * len_keep, D), jnp.float32),
            jax.ShapeDtypeStruct((B, L), jnp.float32),
            jax.ShapeDtypeStruct((B, L), jnp.int32),
        ),
        scratch_types=[
            pltpu.VMEM((n_chunks, chunk), jnp.int32),
            pltpu.VMEM((chunk, D), jnp.float32),
            pltpu.VMEM((chunk, D), jnp.float32),
            pltpu.VMEM((chunk, D), jnp.float32),
            pltpu.VMEM((chunk, D), jnp.float32),
            pltpu.VMEM((L,), jnp.int32),
            pltpu.VMEM((L,), jnp.float32),
            pltpu.SemaphoreType.DMA,
            pltpu.SemaphoreType.DMA,
            pltpu.SemaphoreType.DMA,
            pltpu.SemaphoreType.DMA,
            pltpu.SemaphoreType.DMA,
            pltpu.SemaphoreType.DMA,
            pltpu.SemaphoreType.DMA,
            pltpu.SemaphoreType.DMA,
        ],
    )
    def sc_call(table_hbm, gidx_hbm, restore_hbm,
                out_hbm, mask_hbm, restore_out_hbm,
                idx_v, buf0, buf1, buf2, buf3, rest_v, mask_v,
                g0, g1, g2, g3, w0, w1, w2, w3):
        wid = lax.axis_index("s") * nc + lax.axis_index("c")
        bufs = (buf0, buf1, buf2, buf3)
        gsem = (g0, g1, g2, g3)
        wsem = (w0, w1, w2, w3)
        nb = 4
        base = wid * len_keep

        def start_g(c):
            return pltpu.async_copy(
                table_hbm.at[idx_v.at[c]], bufs[c % nb], gsem[c % nb])

        def start_w(c):
            return pltpu.async_copy(
                bufs[c % nb], out_hbm.at[pl.ds(base + c * chunk, chunk)],
                wsem[c % nb])

        pltpu.sync_copy(gidx_hbm.at[wid], idx_v)
        gcop = [None] * n_chunks
        wcop = [None] * n_chunks
        gcop[0] = start_g(0)
        gcop[1] = start_g(1)

        pltpu.sync_copy(restore_hbm.at[wid], rest_v)
        for i in range(L // LANES):
            r = rest_v[pl.ds(i * LANES, LANES)]
            mask_v[pl.ds(i * LANES, LANES)] = jnp.where(
                r >= len_keep, jnp.float32(1.0), jnp.float32(0.0))
        pltpu.sync_copy(rest_v, restore_out_hbm.at[wid])
        pltpu.sync_copy(mask_v, mask_hbm.at[wid])

        for c in range(n_chunks):
            if c + 2 < n_chunks:
                if c - 2 >= 0:
                    wcop[c - 2].wait()
                gcop[c + 2] = start_g(c + 2)
            gcop[c].wait()
            wcop[c] = start_w(c)
        for j in range(max(0, n_chunks - 4), n_chunks):
            wcop[j].wait()

    return sc_call


def kernel(patches):
    B, L, D = patches.shape
    len_keep = int(L * (1 - RATIO))
    keep_np, restore_np = _plan(B, L, len_keep)

    n_chunks, chunk = 16, len_keep // 16
    gidx_np = (np.arange(B, dtype=np.int64)[:, None] * L
               + keep_np).astype(np.int32).reshape(B, n_chunks, chunk)

    sc_call = _build_sc_call(B, L, D, len_keep, n_chunks, chunk)
    table = patches.reshape(B * L, D)
    out, mask, restore = sc_call(table, jnp.asarray(gidx_np),
                                 jnp.asarray(restore_np))
    return out.reshape(B, len_keep, D), mask, restore


# speedup vs baseline: 2.0671x; 1.0203x over previous
import functools

import jax
import jax.numpy as jnp
import numpy as np
from jax import lax
from jax.experimental import pallas as pl
from jax.experimental.pallas import tpu as pltpu
from jax.experimental.pallas import tpu_sc as plsc

RATIO = 0.5
LANES = 16


def _rotl(x, r):
    return ((x << np.uint32(r)) | (x >> np.uint32(32 - r))).astype(np.uint32)


def _threefry2x32(k0, k1, x0, x1):
    rot = [[13, 15, 26, 6], [17, 29, 16, 24]]
    ks = [np.uint32(k0), np.uint32(k1),
          np.uint32(np.uint32(k0) ^ np.uint32(k1) ^ np.uint32(0x1BD11BDA))]
    x0 = (x0 + ks[0]).astype(np.uint32)
    x1 = (x1 + ks[1]).astype(np.uint32)
    for i in range(5):
        for r in rot[i % 2]:
            x0 = (x0 + x1).astype(np.uint32)
            x1 = _rotl(x1, r)
            x1 = (x1 ^ x0).astype(np.uint32)
        x0 = (x0 + ks[(i + 1) % 3]).astype(np.uint32)
        x1 = (x1 + ks[(i + 2) % 3] + np.uint32(i + 1)).astype(np.uint32)
    return x0, x1


@functools.lru_cache(maxsize=None)
def _plan(B, L, len_keep):
    size = B * L
    c64 = np.arange(size, dtype=np.uint64)
    hi = (c64 >> np.uint64(32)).astype(np.uint32)
    lo = (c64 & np.uint64(0xFFFFFFFF)).astype(np.uint32)
    b0, b1 = _threefry2x32(0, 1, hi, lo)
    bits = (b0 ^ b1).reshape(B, L)
    noise = ((bits >> np.uint32(9)) | np.uint32(0x3F800000)).view(np.float32) - 1.0
    shuffle = np.argsort(noise, axis=1, kind="stable")
    restore = np.argsort(shuffle, axis=1, kind="stable")
    keep = shuffle[:, :len_keep]
    return keep.astype(np.int32), restore.astype(np.int32)


def _build_sc_call(B, L, D, len_keep, n_chunks, chunk):
    info = plsc.get_sparse_core_info()
    nc, ns = info.num_cores, info.num_subcores
    nw = nc * ns
    assert B == nw, (B, nw)
    mesh = plsc.VectorSubcoreMesh(core_axis_name="c", subcore_axis_name="s")

    @functools.partial(
        pl.kernel,
        mesh=mesh,
        out_type=(
            jax.ShapeDtypeStruct((B * len_keep, D), jnp.float32),
            jax.ShapeDtypeStruct((B, L), jnp.float32),
            jax.ShapeDtypeStruct((B, L), jnp.int32),
        ),
        scratch_types=[
            pltpu.VMEM((n_chunks, chunk), jnp.int32),
            pltpu.VMEM((chunk, D), jnp.float32),
            pltpu.VMEM((chunk, D), jnp.float32),
            pltpu.VMEM((L,), jnp.int32),
            pltpu.VMEM((L,), jnp.float32),
            pltpu.SemaphoreType.DMA,
            pltpu.SemaphoreType.DMA,
            pltpu.SemaphoreType.DMA,
            pltpu.SemaphoreType.DMA,
        ],
    )
    def sc_call(table_hbm, gidx_hbm, restore_hbm,
                out_hbm, mask_hbm, restore_out_hbm,
                idx_v, buf0, buf1, rest_v, mask_v,
                gsem0, gsem1, wsem0, wsem1):
        wid = lax.axis_index("s") * nc + lax.axis_index("c")
        base = wid * len_keep

        def start_g(c, buf, sem):
            pltpu.async_copy(table_hbm.at[idx_v.at[c]], buf, sem)

        def wait_g(buf, sem):
            pltpu.make_async_copy(table_hbm.at[pl.ds(0, chunk)], buf, sem).wait()

        def start_w(c, buf, sem):
            pltpu.async_copy(buf, out_hbm.at[pl.ds(base + c * chunk, chunk)], sem)

        def wait_w(buf, sem):
            pltpu.make_async_copy(table_hbm.at[pl.ds(0, chunk)], buf, sem).wait()

        pltpu.sync_copy(gidx_hbm.at[wid], idx_v)
        start_g(0, buf0, gsem0)

        pltpu.sync_copy(restore_hbm.at[wid], rest_v)

        def mask_body(i, carry):
            r = rest_v[pl.ds(i * LANES, LANES)]
            mask_v[pl.ds(i * LANES, LANES)] = jnp.where(
                r >= len_keep, jnp.float32(1.0), jnp.float32(0.0))
            return carry

        lax.fori_loop(0, L // LANES, mask_body, 0, unroll=4)
        pltpu.sync_copy(rest_v, restore_out_hbm.at[wid])
        pltpu.sync_copy(mask_v, mask_hbm.at[wid])

        n2 = n_chunks // 2

        def ring(t, carry):
            @pl.when(t > 0)
            def _():
                wait_w(buf1, wsem1)
            start_g(2 * t + 1, buf1, gsem1)
            wait_g(buf0, gsem0)
            start_w(2 * t, buf0, wsem0)
            wait_w(buf0, wsem0)

            @pl.when(t + 1 < n2)
            def _():
                start_g(2 * t + 2, buf0, gsem0)
            wait_g(buf1, gsem1)
            start_w(2 * t + 1, buf1, wsem1)
            return carry

        lax.fori_loop(0, n2, ring, 0)
        wait_w(buf1, wsem1)

    return sc_call


def kernel(patches):
    B, L, D = patches.shape
    len_keep = int(L * (1 - RATIO))
    keep_np, restore_np = _plan(B, L, len_keep)

    n_chunks, chunk = 8, len_keep // 8
    gidx_np = (np.arange(B, dtype=np.int64)[:, None] * L
               + keep_np).astype(np.int32).reshape(B, n_chunks, chunk)

    sc_call = _build_sc_call(B, L, D, len_keep, n_chunks, chunk)
    table = patches.reshape(B * L, D)
    out, mask, restore = sc_call(table, jnp.asarray(gidx_np),
                                 jnp.asarray(restore_np))
    return out.reshape(B, len_keep, D), mask, restore
